# baseline (speedup 1.0000x reference)
import functools
import math

import jax
import jax.numpy as jnp
from jax import lax
from jax.experimental import pallas as pl
from jax.experimental.pallas import tpu as pltpu
from jax.experimental.pallas import tpu_sc as plsc

_EPS_BN = 1e-5
_INV_SQRT = 1.0 / math.sqrt(1.0 + _EPS_BN)

_CH = 80
_NSUB = 16

_SC_PARAMS = pltpu.CompilerParams(use_tc_tiling_on_sc=False,
                                  needs_layout_passes=False)


def _node_proj_body(h_ref, wn_ref, bn_ref, gn_ref, btn_ref,
                    ah_ref, bh0_ref, bh1_ref, dh0_ref, dh1_ref,
                    th0_ref, th1_ref):
    d = h_ref.shape[1]
    hd = d // 2
    scale = gn_ref[0, :] * _INV_SQRT
    hb = h_ref[...] * scale[None, :] + btn_ref[0, :][None, :]
    H = lax.dot_general(hb, wn_ref[...], (((1,), (1,)), ((), ())),
                        preferred_element_type=jnp.float32)
    H = H + bn_ref[0, :][None, :]
    ah_ref[...] = H[:, :d]
    bh0_ref[...] = H[:, d:d + hd]
    bh1_ref[...] = H[:, d + hd:2 * d]
    dh0_ref[...] = H[:, 2 * d:2 * d + hd]
    dh1_ref[...] = H[:, 2 * d + hd:3 * d]
    th0_ref[...] = H[:, 3 * d:3 * d + hd]
    th1_ref[...] = H[:, 3 * d + hd:4 * d]


def _node_proj(h, Wn, bn, gamma_n, beta_n, row_block=400):
    n, d = h.shape
    hd = d // 2
    grid = (n // row_block,)
    f32 = jnp.float32
    outs = [jax.ShapeDtypeStruct((n, d), f32)] + \
           [jax.ShapeDtypeStruct((n, hd), f32)] * 6
    return pl.pallas_call(
        _node_proj_body,
        grid=grid,
        in_specs=[
            pl.BlockSpec((row_block, d), lambda i: (i, 0)),
            pl.BlockSpec((4 * d, d), lambda i: (0, 0)),
            pl.BlockSpec((1, 4 * d), lambda i: (0, 0)),
            pl.BlockSpec((1, d), lambda i: (0, 0)),
            pl.BlockSpec((1, d), lambda i: (0, 0)),
        ],
        out_specs=[pl.BlockSpec((row_block, d), lambda i: (i, 0))] +
                  [pl.BlockSpec((row_block, hd), lambda i: (i, 0))] * 6,
        out_shape=outs,
    )(h, Wn, bn.reshape(1, -1), gamma_n.reshape(1, -1), beta_n.reshape(1, -1))


def _edge_proj_body(e_ref, we_ref, be_ref, ge_ref, bte_ref, ce_ref):
    scale = ge_ref[0, :] * _INV_SQRT
    eb = e_ref[...] * scale[None, :] + bte_ref[0, :][None, :]
    C = lax.dot_general(eb, we_ref[...], (((1,), (1,)), ((), ())),
                        preferred_element_type=jnp.float32)
    ce_ref[...] = C + be_ref[0, :][None, :]


def _edge_proj(e, We, be, gamma_e, beta_e, row_block=2000):
    m, d = e.shape
    grid = (m // row_block,)
    return pl.pallas_call(
        _edge_proj_body,
        grid=grid,
        in_specs=[
            pl.BlockSpec((row_block, d), lambda i: (i, 0)),
            pl.BlockSpec((d, d), lambda i: (0, 0)),
            pl.BlockSpec((1, d), lambda i: (0, 0)),
            pl.BlockSpec((1, d), lambda i: (0, 0)),
            pl.BlockSpec((1, d), lambda i: (0, 0)),
        ],
        out_specs=pl.BlockSpec((row_block, d), lambda i: (i, 0)),
        out_shape=jax.ShapeDtypeStruct((m, d), jnp.float32),
    )(e, We, be.reshape(1, -1), gamma_e.reshape(1, -1), beta_e.reshape(1, -1))


def _sc_phase1(n, m, d, g16, src, dst, bh0, bh1, dh0, dh1, th0, th1,
               ce, e, zeros_b):
    f32 = jnp.float32
    hd = d // 2
    ept = m // _NSUB
    nch = ept // _CH
    assert nch % 2 == 0 and nch >= 4
    rows_per = (n // (8 * _NSUB)) * 8
    rows_tail = n - rows_per * _NSUB
    nq = hd // 16
    mesh = plsc.VectorSubcoreMesh(core_axis_name="c", subcore_axis_name="s")

    vbuf = pltpu.VMEM((_CH, hd), f32)
    ibuf = pltpu.VMEM((_CH,), jnp.int32)
    dsem = pltpu.SemaphoreType.DMA

    bf16 = jnp.bfloat16
    bbuf = pltpu.VMEM((_CH, hd), bf16)

    @functools.partial(
        pl.kernel,
        mesh=mesh,
        out_type=[jax.ShapeDtypeStruct((m, d), f32),
                  jax.ShapeDtypeStruct((n, hd), bf16),
                  jax.ShapeDtypeStruct((n, hd), bf16),
                  jax.ShapeDtypeStruct((n, hd), bf16),
                  jax.ShapeDtypeStruct((n, hd), bf16)],
        scratch_types=(
            [ibuf] * 6 +
            [vbuf] * 10 +
            [bbuf] * 2 +
            [vbuf] * 2 +
            [bbuf] * 2 +
            [pltpu.VMEM((16,), f32)] +
            [pltpu.VMEM_SHARED((n, hd), bf16)] +
            [pltpu.VMEM_SHARED((n, hd), bf16)] +
            [dsem] * 10
        ),
        compiler_params=_SC_PARAMS,
    )
    def sc_kernel(g_hbm, src_hbm, dst_hbm, bh0_h, bh1_h, dh0_h, dh1_h,
                  th0_h, th1_h, ce_h, e_h, zb_h,
                  eo_h, ph0_h, ph1_h, pp0_h, pp1_h,
                  is0, is1, id0, id1, ic0, ic1,
                  dh_0, dh_1, eh_0, eh_1, bh_0, bh_1, ce_0, ce_1, er_0, er_1,
                  ms_0, ms_1, eo_0, eo_1, sb_0, sb_1,
                  gv, acc, acc_s,
                  si0, si1, sd0, sd1, sw0, sw1, ss0, ss1, sp0, sp1):
        c = lax.axis_index("c")
        s = lax.axis_index("s")
        r0 = s * rows_per
        pltpu.sync_copy(zb_h.at[pl.ds(r0, rows_per)],
                        acc.at[pl.ds(r0, rows_per)])
        pltpu.sync_copy(zb_h.at[pl.ds(r0, rows_per)],
                        acc_s.at[pl.ds(r0, rows_per)])
        if rows_tail:
            @pl.when(s == _NSUB - 1)
            def _():
                t0 = rows_per * _NSUB
                pltpu.sync_copy(zb_h.at[pl.ds(t0, rows_tail)],
                                acc.at[pl.ds(t0, rows_tail)])
                pltpu.sync_copy(zb_h.at[pl.ds(t0, rows_tail)],
                                acc_s.at[pl.ds(t0, rows_tail)])
        pltpu.sync_copy(g_hbm, gv)
        plsc.subcore_barrier()

        IS, ID, IC = [is0, is1], [id0, id1], [ic0, ic1]
        DH, EH, BH = [dh_0, dh_1], [eh_0, eh_1], [bh_0, bh_1]
        CE, ER = [ce_0, ce_1], [er_0, er_1]
        MS, EO, SB = [ms_0, ms_1], [eo_0, eo_1], [sb_0, sb_1]
        SI, SD = [si0, si1], [sd0, sd1]
        SW, SS, SP = [sw0, sw1], [ss0, ss1], [sp0, sp1]

        def run_half(bh_t, dh_t, th_t, ph_t, pp_t, col0):
            gval = gv[...]

            def base_of(gi):
                return s * ept + gi * _CH

            def issue_idx(gi, b):
                bb = base_of(gi)
                pltpu.async_copy(src_hbm.at[pl.ds(bb, _CH)], IS[b], SI[b])
                pltpu.async_copy(dst_hbm.at[pl.ds(bb, _CH)], ID[b], SI[b])

            def wait_idx(b):
                pltpu.make_async_copy(src_hbm.at[pl.ds(0, _CH)], IS[b],
                                      SI[b]).wait()
                pltpu.make_async_copy(dst_hbm.at[pl.ds(0, _CH)], ID[b],
                                      SI[b]).wait()

            def issue_data(gi, b):
                bb = base_of(gi)
                pltpu.async_copy(dh_t.at[IS[b]], DH[b], SD[b])
                pltpu.async_copy(th_t.at[ID[b]], EH[b], SD[b])
                pltpu.async_copy(bh_t.at[IS[b]], BH[b], SD[b])
                pltpu.async_copy(ce_h.at[pl.ds(bb, _CH), pl.ds(col0, hd)],
                                 CE[b], SD[b])
                pltpu.async_copy(e_h.at[pl.ds(bb, _CH), pl.ds(col0, hd)],
                                 ER[b], SD[b])

            def wait_data(b):
                pltpu.make_async_copy(dh_t.at[IS[b]], DH[b], SD[b]).wait()
                pltpu.make_async_copy(th_t.at[ID[b]], EH[b], SD[b]).wait()
                pltpu.make_async_copy(bh_t.at[IS[b]], BH[b], SD[b]).wait()
                pltpu.make_async_copy(
                    ce_h.at[pl.ds(0, _CH), pl.ds(col0, hd)], CE[b],
                    SD[b]).wait()
                pltpu.make_async_copy(
                    e_h.at[pl.ds(0, _CH), pl.ds(col0, hd)], ER[b],
                    SD[b]).wait()

            def copy_idx_sc(b):
                for i in range(_CH // 16):
                    sl = pl.ds(i * 16, 16)
                    IC[b][sl] = ID[b][sl]

            def compute(b):
                @pl.loop(0, _CH)
                def _(j):
                    for qq in range(nq // 2):
                        sgs, mss = [], []
                        for q in (2 * qq, 2 * qq + 1):
                            sl = pl.ds(q * 16, 16)
                            x = DH[b][j, sl] + EH[b][j, sl] + CE[b][j, sl]
                            sgv = 1.0 / (1.0 + jnp.exp(-x))
                            mss.append(BH[b][j, sl] * sgv)
                            EO[b][j, sl] = ER[b][j, sl] + gval * jnp.maximum(
                                x, 0.0)
                            sgs.append(sgv)
                        sl32 = pl.ds(qq * 32, 32)
                        SB[b][j, sl32] = plsc.pack(
                            sgs[0], sgs[1],
                            format=plsc.PackFormat.INTERLEAVED)
                        MS[b][j, sl32] = plsc.pack(
                            mss[0], mss[1],
                            format=plsc.PackFormat.INTERLEAVED)

            def issue_writes(gi, b):
                bb = base_of(gi)
                pltpu.async_copy(EO[b],
                                 eo_h.at[pl.ds(bb, _CH), pl.ds(col0, hd)],
                                 SW[b])
                pltpu.async_copy(MS[b], acc.at[IC[b]], SS[b], add=True)
                pltpu.async_copy(SB[b], acc_s.at[IC[b]], SP[b], add=True)

            def wait_writes(b):
                pltpu.make_async_copy(
                    EO[b], eo_h.at[pl.ds(0, _CH), pl.ds(col0, hd)],
                    SW[b]).wait()
                pltpu.make_async_copy(MS[b], acc.at[IC[b]], SS[b]).wait()
                pltpu.make_async_copy(SB[b], acc_s.at[IC[b]], SP[b]).wait()

            issue_idx(0, 0)
            issue_idx(1, 1)
            wait_idx(0)
            issue_data(0, 0)

            @pl.loop(0, nch // 2)
            def _(t):
                for b in (0, 1):
                    @pl.when(t >= 1)
                    def _():
                        wait_writes(b)
                    wait_data(b)
                    copy_idx_sc(b)

                    @pl.when(t < nch // 2 - 1)
                    def _():
                        issue_idx(2 * t + b + 2, b)

                    if b == 0:
                        wait_idx(1)
                        issue_data(2 * t + 1, 1)
                    else:
                        @pl.when(t < nch // 2 - 1)
                        def _():
                            wait_idx(0)
                            issue_data(2 * t + 2, 0)
                    compute(b)
                    issue_writes(2 * t + b, b)

            wait_writes(0)
            wait_writes(1)

            plsc.subcore_barrier()
            pltpu.sync_copy(acc.at[pl.ds(r0, rows_per)],
                            ph_t.at[pl.ds(r0, rows_per)])
            pltpu.sync_copy(acc_s.at[pl.ds(r0, rows_per)],
                            pp_t.at[pl.ds(r0, rows_per)])
            if rows_tail:
                @pl.when(s == _NSUB - 1)
                def _():
                    t0 = rows_per * _NSUB
                    pltpu.sync_copy(acc.at[pl.ds(t0, rows_tail)],
                                    ph_t.at[pl.ds(t0, rows_tail)])
                    pltpu.sync_copy(acc_s.at[pl.ds(t0, rows_tail)],
                                    pp_t.at[pl.ds(t0, rows_tail)])

        @pl.when(c == 0)
        def _():
            run_half(bh0_h, dh0_h, th0_h, ph0_h, pp0_h, 0)

        @pl.when(c == 1)
        def _():
            run_half(bh1_h, dh1_h, th1_h, ph1_h, pp1_h, hd)

    return sc_kernel(g16, src, dst, bh0, bh1, dh0, dh1, th0, th1,
                     ce, e, zeros_b)


def _node_out_body(h_ref, ah_ref, ph0_ref, ph1_ref, ps0_ref, ps1_ref, g_ref,
                   out_ref):
    d = h_ref.shape[1]
    hd = d // 2
    g = g_ref[0, 0]
    rl = ph0_ref[...] / (ps0_ref[...] + 1e-6)
    rh = ph1_ref[...] / (ps1_ref[...] + 1e-6)
    out_ref[:, :hd] = h_ref[:, :hd] + g * jnp.maximum(ah_ref[:, :hd] + rl,
                                                      0.0)
    out_ref[:, hd:] = h_ref[:, hd:] + g * jnp.maximum(ah_ref[:, hd:] + rh,
                                                      0.0)


def _node_out(h, ah, ph0, ph1, ps0, ps1, g, row_block=400):
    n, d = h.shape
    hd = d // 2
    grid = (n // row_block,)
    spec = pl.BlockSpec((row_block, d), lambda i: (i, 0))
    hspec = pl.BlockSpec((row_block, hd), lambda i: (i, 0))
    return pl.pallas_call(
        _node_out_body,
        grid=grid,
        in_specs=[spec, spec, hspec, hspec, hspec, hspec,
                  pl.BlockSpec((1, 1), lambda i: (0, 0))],
        out_specs=spec,
        out_shape=jax.ShapeDtypeStruct((n, d), jnp.float32),
    )(h, ah, ph0, ph1, ps0, ps1, g.reshape(1, 1))


def kernel(h, e, edge_index, Wn, bn, We, be, gamma_n, beta_n,
           gamma_e, beta_e, g):
    n, d = h.shape
    m = e.shape[0]
    hd = d // 2

    ah, bh0, bh1, dh0, dh1, th0, th1 = _node_proj(h, Wn, bn, gamma_n, beta_n)
    ce = _edge_proj(e, We, be, gamma_e, beta_e)

    src = edge_index[0]
    dst = edge_index[1]

    g16 = jnp.broadcast_to(g.astype(jnp.float32), (16,))
    zeros_b = jnp.zeros((n, hd), jnp.bfloat16)
    e_out, hp0, hp1, pp0, pp1 = _sc_phase1(
        n, m, d, g16, src, dst, bh0, bh1, dh0, dh1, th0, th1, ce, e,
        zeros_b)

    pos = [(col // 32) * 32 + 2 * (col % 16) + (col % 32) // 16
           for col in range(hd)]
    pos = jnp.asarray(pos, jnp.int32)
    ph0 = hp0.astype(jnp.float32)[:, pos]
    ph1 = hp1.astype(jnp.float32)[:, pos]
    ps0 = pp0.astype(jnp.float32)[:, pos]
    ps1 = pp1.astype(jnp.float32)[:, pos]

    h_out = _node_out(h, ah, ph0, ph1, ps0, ps1, g)
    return (h_out, e_out)


# speedup vs baseline: 1.3741x; 1.3741x over previous
import functools
import math

import jax
import jax.numpy as jnp
from jax import lax
from jax.experimental import pallas as pl
from jax.experimental.pallas import tpu as pltpu
from jax.experimental.pallas import tpu_sc as plsc

_EPS_BN = 1e-5
_INV_SQRT = 1.0 / math.sqrt(1.0 + _EPS_BN)

_CH = 80
_NSUB = 16

_SC_PARAMS = pltpu.CompilerParams(use_tc_tiling_on_sc=False)


def _node_proj_body(h_ref, wn_ref, bn_ref, gn_ref, btn_ref,
                    ah_ref, bh0_ref, bh1_ref, dh0_ref, dh1_ref,
                    th0_ref, th1_ref):
    d = h_ref.shape[1]
    hd = d // 2
    scale = gn_ref[0, :] * _INV_SQRT
    hb = h_ref[...] * scale[None, :] + btn_ref[0, :][None, :]
    H = lax.dot_general(hb, wn_ref[...], (((1,), (1,)), ((), ())),
                        preferred_element_type=jnp.float32)
    H = H + bn_ref[0, :][None, :]
    ah_ref[...] = H[:, :d]
    bh0_ref[...] = H[:, d:d + hd]
    bh1_ref[...] = H[:, d + hd:2 * d]
    dh0_ref[...] = H[:, 2 * d:2 * d + hd]
    dh1_ref[...] = H[:, 2 * d + hd:3 * d]
    th0_ref[...] = H[:, 3 * d:3 * d + hd]
    th1_ref[...] = H[:, 3 * d + hd:4 * d]


def _node_proj(h, Wn, bn, gamma_n, beta_n, row_block=400):
    n, d = h.shape
    hd = d // 2
    grid = (n // row_block,)
    f32 = jnp.float32
    outs = [jax.ShapeDtypeStruct((n, d), f32)] + \
           [jax.ShapeDtypeStruct((n, hd), f32)] * 6
    return pl.pallas_call(
        _node_proj_body,
        grid=grid,
        in_specs=[
            pl.BlockSpec((row_block, d), lambda i: (i, 0)),
            pl.BlockSpec((4 * d, d), lambda i: (0, 0)),
            pl.BlockSpec((1, 4 * d), lambda i: (0, 0)),
            pl.BlockSpec((1, d), lambda i: (0, 0)),
            pl.BlockSpec((1, d), lambda i: (0, 0)),
        ],
        out_specs=[pl.BlockSpec((row_block, d), lambda i: (i, 0))] +
                  [pl.BlockSpec((row_block, hd), lambda i: (i, 0))] * 6,
        out_shape=outs,
    )(h, Wn, bn.reshape(1, -1), gamma_n.reshape(1, -1), beta_n.reshape(1, -1))


def _edge_proj_body(e_ref, we_ref, be_ref, ge_ref, bte_ref, ce_ref):
    scale = ge_ref[0, :] * _INV_SQRT
    eb = e_ref[...] * scale[None, :] + bte_ref[0, :][None, :]
    C = lax.dot_general(eb, we_ref[...], (((1,), (1,)), ((), ())),
                        preferred_element_type=jnp.float32)
    ce_ref[...] = C + be_ref[0, :][None, :]


def _edge_proj(e, We, be, gamma_e, beta_e, row_block=2000):
    m, d = e.shape
    grid = (m // row_block,)
    return pl.pallas_call(
        _edge_proj_body,
        grid=grid,
        in_specs=[
            pl.BlockSpec((row_block, d), lambda i: (i, 0)),
            pl.BlockSpec((d, d), lambda i: (0, 0)),
            pl.BlockSpec((1, d), lambda i: (0, 0)),
            pl.BlockSpec((1, d), lambda i: (0, 0)),
            pl.BlockSpec((1, d), lambda i: (0, 0)),
        ],
        out_specs=pl.BlockSpec((row_block, d), lambda i: (i, 0)),
        out_shape=jax.ShapeDtypeStruct((m, d), jnp.float32),
    )(e, We, be.reshape(1, -1), gamma_e.reshape(1, -1), beta_e.reshape(1, -1))


def _sc_phase1(n, m, d, g16, src, dst, bh0, bh1, dh0, dh1, th0, th1,
               ce, e, zeros):
    f32 = jnp.float32
    hd = d // 2
    ept = m // _NSUB
    nch = ept // _CH
    assert nch % 2 == 0 and nch >= 4
    rows_per = (n // (8 * _NSUB)) * 8
    rows_tail = n - rows_per * _NSUB
    nq = hd // 16
    mesh = plsc.VectorSubcoreMesh(core_axis_name="c", subcore_axis_name="s")

    vbuf = pltpu.VMEM((_CH, hd), f32)
    ibuf = pltpu.VMEM((_CH,), jnp.int32)
    dsem = pltpu.SemaphoreType.DMA

    @functools.partial(
        pl.kernel,
        mesh=mesh,
        out_type=[jax.ShapeDtypeStruct((m, d), f32),
                  jax.ShapeDtypeStruct((m, hd), f32),
                  jax.ShapeDtypeStruct((m, hd), f32),
                  jax.ShapeDtypeStruct((n, hd), f32),
                  jax.ShapeDtypeStruct((n, hd), f32)],
        scratch_types=(
            [ibuf] * 6 +
            [vbuf] * 10 +
            [vbuf] * 6 +
            [pltpu.VMEM((16,), f32)] +
            [pltpu.VMEM_SHARED((n, hd), f32)] +
            [dsem] * 8
        ),
        compiler_params=_SC_PARAMS,
    )
    def sc_kernel(g_hbm, src_hbm, dst_hbm, bh0_h, bh1_h, dh0_h, dh1_h,
                  th0_h, th1_h, ce_h, e_h, zz_h,
                  eo_h, sg0_h, sg1_h, ph0_h, ph1_h,
                  is0, is1, id0, id1, ic0, ic1,
                  dh_0, dh_1, eh_0, eh_1, bh_0, bh_1, ce_0, ce_1, er_0, er_1,
                  sg_0, sg_1, ms_0, ms_1, eo_0, eo_1,
                  gv, acc,
                  si0, si1, sd0, sd1, sw0, sw1, ss0, ss1):
        c = lax.axis_index("c")
        s = lax.axis_index("s")
        r0 = s * rows_per
        pltpu.sync_copy(zz_h.at[pl.ds(r0, rows_per)],
                        acc.at[pl.ds(r0, rows_per)])
        if rows_tail:
            @pl.when(s == _NSUB - 1)
            def _():
                t0 = rows_per * _NSUB
                pltpu.sync_copy(zz_h.at[pl.ds(t0, rows_tail)],
                                acc.at[pl.ds(t0, rows_tail)])
        pltpu.sync_copy(g_hbm, gv)
        plsc.subcore_barrier()

        IS, ID, IC = [is0, is1], [id0, id1], [ic0, ic1]
        DH, EH, BH = [dh_0, dh_1], [eh_0, eh_1], [bh_0, bh_1]
        CE, ER = [ce_0, ce_1], [er_0, er_1]
        SG, MS, EO = [sg_0, sg_1], [ms_0, ms_1], [eo_0, eo_1]
        SI, SD = [si0, si1], [sd0, sd1]
        SW, SS = [sw0, sw1], [ss0, ss1]

        def run_half(bh_t, dh_t, th_t, sg_out, ph_t, col0):
            gval = gv[...]

            def base_of(gi):
                return s * ept + gi * _CH

            def issue_idx(gi, b):
                bb = base_of(gi)
                pltpu.async_copy(src_hbm.at[pl.ds(bb, _CH)], IS[b], SI[b])
                pltpu.async_copy(dst_hbm.at[pl.ds(bb, _CH)], ID[b], SI[b])

            def wait_idx(b):
                pltpu.make_async_copy(src_hbm.at[pl.ds(0, _CH)], IS[b],
                                      SI[b]).wait()
                pltpu.make_async_copy(dst_hbm.at[pl.ds(0, _CH)], ID[b],
                                      SI[b]).wait()

            def issue_data(gi, b):
                bb = base_of(gi)
                pltpu.async_copy(dh_t.at[IS[b]], DH[b], SD[b])
                pltpu.async_copy(th_t.at[ID[b]], EH[b], SD[b])
                pltpu.async_copy(bh_t.at[IS[b]], BH[b], SD[b])
                pltpu.async_copy(ce_h.at[pl.ds(bb, _CH), pl.ds(col0, hd)],
                                 CE[b], SD[b])
                pltpu.async_copy(e_h.at[pl.ds(bb, _CH), pl.ds(col0, hd)],
                                 ER[b], SD[b])

            def wait_data(b):
                pltpu.make_async_copy(dh_t.at[IS[b]], DH[b], SD[b]).wait()
                pltpu.make_async_copy(th_t.at[ID[b]], EH[b], SD[b]).wait()
                pltpu.make_async_copy(bh_t.at[IS[b]], BH[b], SD[b]).wait()
                pltpu.make_async_copy(
                    ce_h.at[pl.ds(0, _CH), pl.ds(col0, hd)], CE[b],
                    SD[b]).wait()
                pltpu.make_async_copy(
                    e_h.at[pl.ds(0, _CH), pl.ds(col0, hd)], ER[b],
                    SD[b]).wait()

            def copy_idx_sc(b):
                for i in range(_CH // 16):
                    sl = pl.ds(i * 16, 16)
                    IC[b][sl] = ID[b][sl]

            def compute(b):
                @pl.loop(0, _CH, step=2)
                def _(j0):
                    for dj in range(2):
                        j = j0 + dj
                        for q in range(nq):
                            sl = pl.ds(q * 16, 16)
                            x = DH[b][j, sl] + EH[b][j, sl] + CE[b][j, sl]
                            sgv = 1.0 / (1.0 + jnp.exp(-x))
                            SG[b][j, sl] = sgv
                            MS[b][j, sl] = BH[b][j, sl] * sgv
                            EO[b][j, sl] = ER[b][j, sl] + gval * jnp.maximum(
                                x, 0.0)

            def issue_writes(gi, b):
                bb = base_of(gi)
                pltpu.async_copy(EO[b],
                                 eo_h.at[pl.ds(bb, _CH), pl.ds(col0, hd)],
                                 SW[b])
                pltpu.async_copy(SG[b], sg_out.at[pl.ds(bb, _CH)], SW[b])
                pltpu.async_copy(MS[b], acc.at[IC[b]], SS[b], add=True)

            def wait_writes(b):
                pltpu.make_async_copy(
                    EO[b], eo_h.at[pl.ds(0, _CH), pl.ds(col0, hd)],
                    SW[b]).wait()
                pltpu.make_async_copy(SG[b], sg_out.at[pl.ds(0, _CH)],
                                      SW[b]).wait()
                pltpu.make_async_copy(MS[b], acc.at[IC[b]], SS[b]).wait()

            issue_idx(0, 0)
            issue_idx(1, 1)
            wait_idx(0)
            issue_data(0, 0)

            @pl.loop(0, nch // 2)
            def _(t):
                for b in (0, 1):
                    @pl.when(t >= 1)
                    def _():
                        wait_writes(b)
                    wait_data(b)
                    copy_idx_sc(b)

                    @pl.when(t < nch // 2 - 1)
                    def _():
                        issue_idx(2 * t + b + 2, b)

                    if b == 0:
                        wait_idx(1)
                        issue_data(2 * t + 1, 1)
                    else:
                        @pl.when(t < nch // 2 - 1)
                        def _():
                            wait_idx(0)
                            issue_data(2 * t + 2, 0)
                    compute(b)
                    issue_writes(2 * t + b, b)

            wait_writes(0)
            wait_writes(1)

            plsc.subcore_barrier()
            pltpu.sync_copy(acc.at[pl.ds(r0, rows_per)],
                            ph_t.at[pl.ds(r0, rows_per)])
            if rows_tail:
                @pl.when(s == _NSUB - 1)
                def _():
                    t0 = rows_per * _NSUB
                    pltpu.sync_copy(acc.at[pl.ds(t0, rows_tail)],
                                    ph_t.at[pl.ds(t0, rows_tail)])

        @pl.when(c == 0)
        def _():
            run_half(bh0_h, dh0_h, th0_h, sg0_h, ph0_h, 0)

        @pl.when(c == 1)
        def _():
            run_half(bh1_h, dh1_h, th1_h, sg1_h, ph1_h, hd)

    return sc_kernel(g16, src, dst, bh0, bh1, dh0, dh1, th0, th1,
                     ce, e, zeros)


def _sc_phase2(n, m, d, dst, sg0, sg1, zeros):
    f32 = jnp.float32
    hd = d // 2
    ept = m // _NSUB
    nch = ept // _CH
    rows_per = (n // (8 * _NSUB)) * 8
    rows_tail = n - rows_per * _NSUB
    mesh = plsc.VectorSubcoreMesh(core_axis_name="c", subcore_axis_name="s")

    ch2 = _CH // 2
    nch2 = ept // ch2
    assert nch2 % 4 == 0 and nch2 >= 8
    ibuf = pltpu.VMEM((ch2,), jnp.int32)
    vbuf = pltpu.VMEM((ch2, hd), f32)
    dsem = pltpu.SemaphoreType.DMA

    @functools.partial(
        pl.kernel,
        mesh=mesh,
        out_type=[jax.ShapeDtypeStruct((n, hd), f32),
                  jax.ShapeDtypeStruct((n, hd), f32)],
        scratch_types=(
            [ibuf] * 4 +
            [vbuf] * 4 +
            [pltpu.VMEM_SHARED((n, hd), f32)] +
            [dsem] * 8
        ),
        compiler_params=_SC_PARAMS,
    )
    def sc_kernel(dst_hbm, sg0_h, sg1_h, zz_h, ps0_h, ps1_h,
                  id0, id1, id2, id3, sg_0, sg_1, sg_2, sg_3, acc,
                  sl0, sl1, sl2, sl3, ss0, ss1, ss2, ss3):
        c = lax.axis_index("c")
        s = lax.axis_index("s")
        r0 = s * rows_per
        pltpu.sync_copy(zz_h.at[pl.ds(r0, rows_per)],
                        acc.at[pl.ds(r0, rows_per)])
        if rows_tail:
            @pl.when(s == _NSUB - 1)
            def _():
                t0 = rows_per * _NSUB
                pltpu.sync_copy(zz_h.at[pl.ds(t0, rows_tail)],
                                acc.at[pl.ds(t0, rows_tail)])
        plsc.subcore_barrier()

        ID = [id0, id1, id2, id3]
        SGR = [sg_0, sg_1, sg_2, sg_3]
        SL = [sl0, sl1, sl2, sl3]
        SS = [ss0, ss1, ss2, ss3]

        def run_half(sg_t, ps_t):
            def issue_loads(gi, b):
                base = s * ept + gi * ch2
                pltpu.async_copy(dst_hbm.at[pl.ds(base, ch2)], ID[b], SL[b])
                pltpu.async_copy(sg_t.at[pl.ds(base, ch2)], SGR[b], SL[b])

            def wait_loads(b):
                pltpu.make_async_copy(dst_hbm.at[pl.ds(0, ch2)], ID[b],
                                      SL[b]).wait()
                pltpu.make_async_copy(sg_t.at[pl.ds(0, ch2)], SGR[b],
                                      SL[b]).wait()

            def wait_scatter(b):
                pltpu.make_async_copy(SGR[b], acc.at[ID[b]], SS[b]).wait()

            issue_loads(0, 0)
            issue_loads(1, 1)

            @pl.loop(0, nch2 // 4)
            def _(t):
                for b in range(4):
                    wait_loads(b)
                    pltpu.async_copy(SGR[b], acc.at[ID[b]], SS[b],
                                     add=True)
                    nb2 = (b + 2) % 4
                    if b < 2:
                        @pl.when(t >= 1)
                        def _():
                            wait_scatter(nb2)
                        issue_loads(4 * t + b + 2, nb2)
                    else:
                        wait_scatter(nb2)

                        @pl.when(t < nch2 // 4 - 1)
                        def _():
                            issue_loads(4 * t + b + 2, nb2)

            wait_scatter((nch2 - 2) % 4)
            wait_scatter((nch2 - 1) % 4)

            plsc.subcore_barrier()
            pltpu.sync_copy(acc.at[pl.ds(r0, rows_per)],
                            ps_t.at[pl.ds(r0, rows_per)])
            if rows_tail:
                @pl.when(s == _NSUB - 1)
                def _():
                    t0 = rows_per * _NSUB
                    pltpu.sync_copy(acc.at[pl.ds(t0, rows_tail)],
                                    ps_t.at[pl.ds(t0, rows_tail)])

        @pl.when(c == 0)
        def _():
            run_half(sg0_h, ps0_h)

        @pl.when(c == 1)
        def _():
            run_half(sg1_h, ps1_h)

    return sc_kernel(dst, sg0, sg1, zeros)


def _node_out_body(h_ref, ah_ref, ph0_ref, ph1_ref, ps0_ref, ps1_ref, g_ref,
                   out_ref):
    d = h_ref.shape[1]
    hd = d // 2
    g = g_ref[0, 0]
    rl = ph0_ref[...] / (ps0_ref[...] + 1e-6)
    rh = ph1_ref[...] / (ps1_ref[...] + 1e-6)
    out_ref[:, :hd] = h_ref[:, :hd] + g * jnp.maximum(ah_ref[:, :hd] + rl,
                                                      0.0)
    out_ref[:, hd:] = h_ref[:, hd:] + g * jnp.maximum(ah_ref[:, hd:] + rh,
                                                      0.0)


def _node_out(h, ah, ph0, ph1, ps0, ps1, g, row_block=400):
    n, d = h.shape
    hd = d // 2
    grid = (n // row_block,)
    spec = pl.BlockSpec((row_block, d), lambda i: (i, 0))
    hspec = pl.BlockSpec((row_block, hd), lambda i: (i, 0))
    return pl.pallas_call(
        _node_out_body,
        grid=grid,
        in_specs=[spec, spec, hspec, hspec, hspec, hspec,
                  pl.BlockSpec((1, 1), lambda i: (0, 0))],
        out_specs=spec,
        out_shape=jax.ShapeDtypeStruct((n, d), jnp.float32),
    )(h, ah, ph0, ph1, ps0, ps1, g.reshape(1, 1))


def kernel(h, e, edge_index, Wn, bn, We, be, gamma_n, beta_n,
           gamma_e, beta_e, g):
    n, d = h.shape
    m = e.shape[0]
    hd = d // 2

    ah, bh0, bh1, dh0, dh1, th0, th1 = _node_proj(h, Wn, bn, gamma_n, beta_n)
    ce = _edge_proj(e, We, be, gamma_e, beta_e)

    src = edge_index[0]
    dst = edge_index[1]

    g16 = jnp.broadcast_to(g.astype(jnp.float32), (16,))
    zeros = jnp.zeros((n, hd), jnp.float32)
    e_out, sg0, sg1, ph0, ph1 = _sc_phase1(
        n, m, d, g16, src, dst, bh0, bh1, dh0, dh1, th0, th1, ce, e, zeros)
    ps0, ps1 = _sc_phase2(n, m, d, dst, sg0, sg1, zeros)

    h_out = _node_out(h, ah, ph0, ph1, ps0, ps1, g)
    return (h_out, e_out)


# speedup vs baseline: 1.4880x; 1.0829x over previous
import functools
import math

import jax
import jax.numpy as jnp
from jax import lax
from jax.experimental import pallas as pl
from jax.experimental.pallas import tpu as pltpu
from jax.experimental.pallas import tpu_sc as plsc

_EPS_BN = 1e-5
_INV_SQRT = 1.0 / math.sqrt(1.0 + _EPS_BN)

_CH = 80
_NSUB = 16

_SC_PARAMS = pltpu.CompilerParams(use_tc_tiling_on_sc=False)


def _node_proj_body(h_ref, wn_ref, bn_ref, gn_ref, btn_ref,
                    ah_ref, bh0_ref, bh1_ref, dh0_ref, dh1_ref,
                    th0_ref, th1_ref):
    d = h_ref.shape[1]
    hd = d // 2
    scale = gn_ref[0, :] * _INV_SQRT
    hb = h_ref[...] * scale[None, :] + btn_ref[0, :][None, :]
    H = lax.dot_general(hb, wn_ref[...], (((1,), (1,)), ((), ())),
                        preferred_element_type=jnp.float32)
    H = H + bn_ref[0, :][None, :]
    ah_ref[...] = H[:, :d]
    bh0_ref[...] = H[:, d:d + hd]
    bh1_ref[...] = H[:, d + hd:2 * d]
    dh0_ref[...] = H[:, 2 * d:2 * d + hd]
    dh1_ref[...] = H[:, 2 * d + hd:3 * d]
    th0_ref[...] = H[:, 3 * d:3 * d + hd]
    th1_ref[...] = H[:, 3 * d + hd:4 * d]


def _node_proj(h, Wn, bn, gamma_n, beta_n, row_block=2000):
    n, d = h.shape
    hd = d // 2
    grid = (n // row_block,)
    f32 = jnp.float32
    outs = [jax.ShapeDtypeStruct((n, d), f32)] + \
           [jax.ShapeDtypeStruct((n, hd), f32)] * 6
    return pl.pallas_call(
        _node_proj_body,
        grid=grid,
        in_specs=[
            pl.BlockSpec((row_block, d), lambda i: (i, 0)),
            pl.BlockSpec((4 * d, d), lambda i: (0, 0)),
            pl.BlockSpec((1, 4 * d), lambda i: (0, 0)),
            pl.BlockSpec((1, d), lambda i: (0, 0)),
            pl.BlockSpec((1, d), lambda i: (0, 0)),
        ],
        out_specs=[pl.BlockSpec((row_block, d), lambda i: (i, 0))] +
                  [pl.BlockSpec((row_block, hd), lambda i: (i, 0))] * 6,
        out_shape=outs,
    )(h, Wn, bn.reshape(1, -1), gamma_n.reshape(1, -1), beta_n.reshape(1, -1))


def _edge_proj_body(e_ref, we_ref, be_ref, ge_ref, bte_ref, ce_ref):
    scale = ge_ref[0, :] * _INV_SQRT
    eb = e_ref[...] * scale[None, :] + bte_ref[0, :][None, :]
    C = lax.dot_general(eb, we_ref[...], (((1,), (1,)), ((), ())),
                        preferred_element_type=jnp.float32)
    ce_ref[...] = C + be_ref[0, :][None, :]


def _edge_proj(e, We, be, gamma_e, beta_e, row_block=4000):
    m, d = e.shape
    grid = (m // row_block,)
    return pl.pallas_call(
        _edge_proj_body,
        grid=grid,
        in_specs=[
            pl.BlockSpec((row_block, d), lambda i: (i, 0)),
            pl.BlockSpec((d, d), lambda i: (0, 0)),
            pl.BlockSpec((1, d), lambda i: (0, 0)),
            pl.BlockSpec((1, d), lambda i: (0, 0)),
            pl.BlockSpec((1, d), lambda i: (0, 0)),
        ],
        out_specs=pl.BlockSpec((row_block, d), lambda i: (i, 0)),
        out_shape=jax.ShapeDtypeStruct((m, d), jnp.float32),
    )(e, We, be.reshape(1, -1), gamma_e.reshape(1, -1), beta_e.reshape(1, -1))


def _sc_phase1(n, m, d, g16, src, dst, bh0, bh1, dh0, dh1, th0, th1,
               ce, e, zeros):
    f32 = jnp.float32
    hd = d // 2
    ept = m // _NSUB
    nch = ept // _CH
    assert nch % 2 == 0 and nch >= 4
    rows_per = (n // (8 * _NSUB)) * 8
    rows_tail = n - rows_per * _NSUB
    nq = hd // 16
    mesh = plsc.VectorSubcoreMesh(core_axis_name="c", subcore_axis_name="s")

    vbuf = pltpu.VMEM((_CH, hd), f32)
    ibuf = pltpu.VMEM((_CH,), jnp.int32)
    dsem = pltpu.SemaphoreType.DMA

    @functools.partial(
        pl.kernel,
        mesh=mesh,
        out_type=[jax.ShapeDtypeStruct((m, d), f32),
                  jax.ShapeDtypeStruct((m, hd), f32),
                  jax.ShapeDtypeStruct((m, hd), f32),
                  jax.ShapeDtypeStruct((n, hd), f32),
                  jax.ShapeDtypeStruct((n, hd), f32)],
        scratch_types=(
            [ibuf] * 6 +
            [vbuf] * 10 +
            [vbuf] * 6 +
            [pltpu.VMEM((16,), f32)] +
            [pltpu.VMEM_SHARED((n, hd), f32)] +
            [dsem] * 8
        ),
        compiler_params=_SC_PARAMS,
    )
    def sc_kernel(g_hbm, src_hbm, dst_hbm, bh0_h, bh1_h, dh0_h, dh1_h,
                  th0_h, th1_h, ce_h, e_h, zz_h,
                  eo_h, sg0_h, sg1_h, ph0_h, ph1_h,
                  is0, is1, id0, id1, ic0, ic1,
                  dh_0, dh_1, eh_0, eh_1, bh_0, bh_1, ce_0, ce_1, er_0, er_1,
                  sg_0, sg_1, ms_0, ms_1, eo_0, eo_1,
                  gv, acc,
                  si0, si1, sd0, sd1, sw0, sw1, ss0, ss1):
        c = lax.axis_index("c")
        s = lax.axis_index("s")
        r0 = s * rows_per
        pltpu.sync_copy(zz_h.at[pl.ds(r0, rows_per)],
                        acc.at[pl.ds(r0, rows_per)])
        if rows_tail:
            @pl.when(s == _NSUB - 1)
            def _():
                t0 = rows_per * _NSUB
                pltpu.sync_copy(zz_h.at[pl.ds(t0, rows_tail)],
                                acc.at[pl.ds(t0, rows_tail)])
        pltpu.sync_copy(g_hbm, gv)
        plsc.subcore_barrier()

        IS, ID, IC = [is0, is1], [id0, id1], [ic0, ic1]
        DH, EH, BH = [dh_0, dh_1], [eh_0, eh_1], [bh_0, bh_1]
        CE, ER = [ce_0, ce_1], [er_0, er_1]
        SG, MS, EO = [sg_0, sg_1], [ms_0, ms_1], [eo_0, eo_1]
        SI, SD = [si0, si1], [sd0, sd1]
        SW, SS = [sw0, sw1], [ss0, ss1]

        def run_half(bh_t, dh_t, th_t, sg_out, ph_t, col0):
            gval = gv[...]

            def base_of(gi):
                return s * ept + gi * _CH

            def issue_idx(gi, b):
                bb = base_of(gi)
                pltpu.async_copy(src_hbm.at[pl.ds(bb, _CH)], IS[b], SI[b])
                pltpu.async_copy(dst_hbm.at[pl.ds(bb, _CH)], ID[b], SI[b])

            def wait_idx(b):
                pltpu.make_async_copy(src_hbm.at[pl.ds(0, _CH)], IS[b],
                                      SI[b]).wait()
                pltpu.make_async_copy(dst_hbm.at[pl.ds(0, _CH)], ID[b],
                                      SI[b]).wait()

            def issue_data(gi, b):
                bb = base_of(gi)
                pltpu.async_copy(dh_t.at[IS[b]], DH[b], SD[b])
                pltpu.async_copy(th_t.at[ID[b]], EH[b], SD[b])
                pltpu.async_copy(bh_t.at[IS[b]], BH[b], SD[b])
                pltpu.async_copy(ce_h.at[pl.ds(bb, _CH), pl.ds(col0, hd)],
                                 CE[b], SD[b])
                pltpu.async_copy(e_h.at[pl.ds(bb, _CH), pl.ds(col0, hd)],
                                 ER[b], SD[b])

            def wait_data(b):
                pltpu.make_async_copy(dh_t.at[IS[b]], DH[b], SD[b]).wait()
                pltpu.make_async_copy(th_t.at[ID[b]], EH[b], SD[b]).wait()
                pltpu.make_async_copy(bh_t.at[IS[b]], BH[b], SD[b]).wait()
                pltpu.make_async_copy(
                    ce_h.at[pl.ds(0, _CH), pl.ds(col0, hd)], CE[b],
                    SD[b]).wait()
                pltpu.make_async_copy(
                    e_h.at[pl.ds(0, _CH), pl.ds(col0, hd)], ER[b],
                    SD[b]).wait()

            def copy_idx_sc(b):
                for i in range(_CH // 16):
                    sl = pl.ds(i * 16, 16)
                    IC[b][sl] = ID[b][sl]

            def compute(b):
                @pl.loop(0, _CH)
                def _(j):
                    for q in range(nq):
                        sl = pl.ds(q * 16, 16)
                        x = DH[b][j, sl] + EH[b][j, sl] + CE[b][j, sl]
                        sgv = 1.0 / (1.0 + jnp.exp(-x))
                        SG[b][j, sl] = sgv
                        MS[b][j, sl] = BH[b][j, sl] * sgv
                        EO[b][j, sl] = ER[b][j, sl] + gval * jnp.maximum(
                            x, 0.0)

            def issue_writes(gi, b):
                bb = base_of(gi)
                pltpu.async_copy(EO[b],
                                 eo_h.at[pl.ds(bb, _CH), pl.ds(col0, hd)],
                                 SW[b])
                pltpu.async_copy(SG[b], sg_out.at[pl.ds(bb, _CH)], SW[b])
                pltpu.async_copy(MS[b], acc.at[IC[b]], SS[b], add=True)

            def wait_writes(b):
                pltpu.make_async_copy(
                    EO[b], eo_h.at[pl.ds(0, _CH), pl.ds(col0, hd)],
                    SW[b]).wait()
                pltpu.make_async_copy(SG[b], sg_out.at[pl.ds(0, _CH)],
                                      SW[b]).wait()
                pltpu.make_async_copy(MS[b], acc.at[IC[b]], SS[b]).wait()

            issue_idx(0, 0)
            issue_idx(1, 1)
            wait_idx(0)
            issue_data(0, 0)

            @pl.loop(0, nch // 2)
            def _(t):
                for b in (0, 1):
                    @pl.when(t >= 1)
                    def _():
                        wait_writes(b)
                    wait_data(b)
                    copy_idx_sc(b)

                    @pl.when(t < nch // 2 - 1)
                    def _():
                        issue_idx(2 * t + b + 2, b)

                    if b == 0:
                        wait_idx(1)
                        issue_data(2 * t + 1, 1)
                    else:
                        @pl.when(t < nch // 2 - 1)
                        def _():
                            wait_idx(0)
                            issue_data(2 * t + 2, 0)
                    compute(b)
                    issue_writes(2 * t + b, b)

            wait_writes(0)
            wait_writes(1)

            plsc.subcore_barrier()
            pltpu.sync_copy(acc.at[pl.ds(r0, rows_per)],
                            ph_t.at[pl.ds(r0, rows_per)])
            if rows_tail:
                @pl.when(s == _NSUB - 1)
                def _():
                    t0 = rows_per * _NSUB
                    pltpu.sync_copy(acc.at[pl.ds(t0, rows_tail)],
                                    ph_t.at[pl.ds(t0, rows_tail)])

        @pl.when(c == 0)
        def _():
            run_half(bh0_h, dh0_h, th0_h, sg0_h, ph0_h, 0)

        @pl.when(c == 1)
        def _():
            run_half(bh1_h, dh1_h, th1_h, sg1_h, ph1_h, hd)

    return sc_kernel(g16, src, dst, bh0, bh1, dh0, dh1, th0, th1,
                     ce, e, zeros)


def _sc_phase2(n, m, d, dst, sg0, sg1, zeros):
    f32 = jnp.float32
    hd = d // 2
    ept = m // _NSUB
    nch = ept // _CH
    rows_per = (n // (8 * _NSUB)) * 8
    rows_tail = n - rows_per * _NSUB
    mesh = plsc.VectorSubcoreMesh(core_axis_name="c", subcore_axis_name="s")

    ch2 = _CH // 2
    nch2 = ept // ch2
    assert nch2 % 4 == 0 and nch2 >= 8
    ibuf = pltpu.VMEM((ch2,), jnp.int32)
    vbuf = pltpu.VMEM((ch2, hd), f32)
    dsem = pltpu.SemaphoreType.DMA

    @functools.partial(
        pl.kernel,
        mesh=mesh,
        out_type=[jax.ShapeDtypeStruct((n, hd), f32),
                  jax.ShapeDtypeStruct((n, hd), f32)],
        scratch_types=(
            [ibuf] * 4 +
            [vbuf] * 4 +
            [pltpu.VMEM_SHARED((n, hd), f32)] +
            [dsem] * 8
        ),
        compiler_params=_SC_PARAMS,
    )
    def sc_kernel(dst_hbm, sg0_h, sg1_h, zz_h, ps0_h, ps1_h,
                  id0, id1, id2, id3, sg_0, sg_1, sg_2, sg_3, acc,
                  sl0, sl1, sl2, sl3, ss0, ss1, ss2, ss3):
        c = lax.axis_index("c")
        s = lax.axis_index("s")
        r0 = s * rows_per
        pltpu.sync_copy(zz_h.at[pl.ds(r0, rows_per)],
                        acc.at[pl.ds(r0, rows_per)])
        if rows_tail:
            @pl.when(s == _NSUB - 1)
            def _():
                t0 = rows_per * _NSUB
                pltpu.sync_copy(zz_h.at[pl.ds(t0, rows_tail)],
                                acc.at[pl.ds(t0, rows_tail)])
        plsc.subcore_barrier()

        ID = [id0, id1, id2, id3]
        SGR = [sg_0, sg_1, sg_2, sg_3]
        SL = [sl0, sl1, sl2, sl3]
        SS = [ss0, ss1, ss2, ss3]

        def run_half(sg_t, ps_t):
            def issue_loads(gi, b):
                base = s * ept + gi * ch2
                pltpu.async_copy(dst_hbm.at[pl.ds(base, ch2)], ID[b], SL[b])
                pltpu.async_copy(sg_t.at[pl.ds(base, ch2)], SGR[b], SL[b])

            def wait_loads(b):
                pltpu.make_async_copy(dst_hbm.at[pl.ds(0, ch2)], ID[b],
                                      SL[b]).wait()
                pltpu.make_async_copy(sg_t.at[pl.ds(0, ch2)], SGR[b],
                                      SL[b]).wait()

            def wait_scatter(b):
                pltpu.make_async_copy(SGR[b], acc.at[ID[b]], SS[b]).wait()

            issue_loads(0, 0)
            issue_loads(1, 1)

            @pl.loop(0, nch2 // 4)
            def _(t):
                for b in range(4):
                    wait_loads(b)
                    pltpu.async_copy(SGR[b], acc.at[ID[b]], SS[b],
                                     add=True)
                    nb2 = (b + 2) % 4
                    if b < 2:
                        @pl.when(t >= 1)
                        def _():
                            wait_scatter(nb2)
                        issue_loads(4 * t + b + 2, nb2)
                    else:
                        wait_scatter(nb2)

                        @pl.when(t < nch2 // 4 - 1)
                        def _():
                            issue_loads(4 * t + b + 2, nb2)

            wait_scatter((nch2 - 2) % 4)
            wait_scatter((nch2 - 1) % 4)

            plsc.subcore_barrier()
            pltpu.sync_copy(acc.at[pl.ds(r0, rows_per)],
                            ps_t.at[pl.ds(r0, rows_per)])
            if rows_tail:
                @pl.when(s == _NSUB - 1)
                def _():
                    t0 = rows_per * _NSUB
                    pltpu.sync_copy(acc.at[pl.ds(t0, rows_tail)],
                                    ps_t.at[pl.ds(t0, rows_tail)])

        @pl.when(c == 0)
        def _():
            run_half(sg0_h, ps0_h)

        @pl.when(c == 1)
        def _():
            run_half(sg1_h, ps1_h)

    return sc_kernel(dst, sg0, sg1, zeros)


def _node_out_body(h_ref, ah_ref, ph0_ref, ph1_ref, ps0_ref, ps1_ref, g_ref,
                   out_ref):
    d = h_ref.shape[1]
    hd = d // 2
    g = g_ref[0, 0]
    rl = ph0_ref[...] / (ps0_ref[...] + 1e-6)
    rh = ph1_ref[...] / (ps1_ref[...] + 1e-6)
    out_ref[:, :hd] = h_ref[:, :hd] + g * jnp.maximum(ah_ref[:, :hd] + rl,
                                                      0.0)
    out_ref[:, hd:] = h_ref[:, hd:] + g * jnp.maximum(ah_ref[:, hd:] + rh,
                                                      0.0)


def _node_out(h, ah, ph0, ph1, ps0, ps1, g, row_block=2000):
    n, d = h.shape
    hd = d // 2
    grid = (n // row_block,)
    spec = pl.BlockSpec((row_block, d), lambda i: (i, 0))
    hspec = pl.BlockSpec((row_block, hd), lambda i: (i, 0))
    return pl.pallas_call(
        _node_out_body,
        grid=grid,
        in_specs=[spec, spec, hspec, hspec, hspec, hspec,
                  pl.BlockSpec((1, 1), lambda i: (0, 0))],
        out_specs=spec,
        out_shape=jax.ShapeDtypeStruct((n, d), jnp.float32),
    )(h, ah, ph0, ph1, ps0, ps1, g.reshape(1, 1))


def kernel(h, e, edge_index, Wn, bn, We, be, gamma_n, beta_n,
           gamma_e, beta_e, g):
    n, d = h.shape
    m = e.shape[0]
    hd = d // 2

    ah, bh0, bh1, dh0, dh1, th0, th1 = _node_proj(h, Wn, bn, gamma_n, beta_n)
    ce = _edge_proj(e, We, be, gamma_e, beta_e)

    src = edge_index[0]
    dst = edge_index[1]

    g16 = jnp.broadcast_to(g.astype(jnp.float32), (16,))
    zeros = jnp.zeros((n, hd), jnp.float32)
    e_out, sg0, sg1, ph0, ph1 = _sc_phase1(
        n, m, d, g16, src, dst, bh0, bh1, dh0, dh1, th0, th1, ce, e, zeros)
    ps0, ps1 = _sc_phase2(n, m, d, dst, sg0, sg1, zeros)

    h_out = _node_out(h, ah, ph0, ph1, ps0, ps1, g)
    return (h_out, e_out)


# speedup vs baseline: 1.5186x; 1.0205x over previous
import functools
import math

import jax
import jax.numpy as jnp
from jax import lax
from jax.experimental import pallas as pl
from jax.experimental.pallas import tpu as pltpu
from jax.experimental.pallas import tpu_sc as plsc

_EPS_BN = 1e-5
_INV_SQRT = 1.0 / math.sqrt(1.0 + _EPS_BN)

_CH = 80
_NSUB = 16

_SC_PARAMS = pltpu.CompilerParams(use_tc_tiling_on_sc=False)


def _node_proj_body(h_ref, wn_ref, bn_ref, gn_ref, btn_ref,
                    ah_ref, bh0_ref, bh1_ref, dh0_ref, dh1_ref,
                    th0_ref, th1_ref):
    d = h_ref.shape[1]
    hd = d // 2
    scale = gn_ref[0, :] * _INV_SQRT
    hb = h_ref[...] * scale[None, :] + btn_ref[0, :][None, :]
    H = lax.dot_general(hb, wn_ref[...], (((1,), (1,)), ((), ())),
                        preferred_element_type=jnp.float32)
    H = H + bn_ref[0, :][None, :]
    ah_ref[...] = H[:, :d]
    bh0_ref[...] = H[:, d:d + hd]
    bh1_ref[...] = H[:, d + hd:2 * d]
    dh0_ref[...] = H[:, 2 * d:2 * d + hd]
    dh1_ref[...] = H[:, 2 * d + hd:3 * d]
    th0_ref[...] = H[:, 3 * d:3 * d + hd]
    th1_ref[...] = H[:, 3 * d + hd:4 * d]


def _node_proj(h, Wn, bn, gamma_n, beta_n, row_block=2000):
    n, d = h.shape
    hd = d // 2
    grid = (n // row_block,)
    f32 = jnp.float32
    outs = [jax.ShapeDtypeStruct((n, d), f32)] + \
           [jax.ShapeDtypeStruct((n, hd), f32)] * 6
    return pl.pallas_call(
        _node_proj_body,
        grid=grid,
        in_specs=[
            pl.BlockSpec((row_block, d), lambda i: (i, 0)),
            pl.BlockSpec((4 * d, d), lambda i: (0, 0)),
            pl.BlockSpec((1, 4 * d), lambda i: (0, 0)),
            pl.BlockSpec((1, d), lambda i: (0, 0)),
            pl.BlockSpec((1, d), lambda i: (0, 0)),
        ],
        out_specs=[pl.BlockSpec((row_block, d), lambda i: (i, 0))] +
                  [pl.BlockSpec((row_block, hd), lambda i: (i, 0))] * 6,
        out_shape=outs,
    )(h, Wn, bn.reshape(1, -1), gamma_n.reshape(1, -1), beta_n.reshape(1, -1))


def _edge_proj_body(e_ref, we_ref, be_ref, ge_ref, bte_ref, ce_ref):
    scale = ge_ref[0, :] * _INV_SQRT
    eb = e_ref[...] * scale[None, :] + bte_ref[0, :][None, :]
    C = lax.dot_general(eb, we_ref[...], (((1,), (1,)), ((), ())),
                        preferred_element_type=jnp.float32)
    ce_ref[...] = C + be_ref[0, :][None, :]


def _edge_proj(e, We, be, gamma_e, beta_e, row_block=8000):
    m, d = e.shape
    grid = (m // row_block,)
    return pl.pallas_call(
        _edge_proj_body,
        grid=grid,
        in_specs=[
            pl.BlockSpec((row_block, d), lambda i: (i, 0)),
            pl.BlockSpec((d, d), lambda i: (0, 0)),
            pl.BlockSpec((1, d), lambda i: (0, 0)),
            pl.BlockSpec((1, d), lambda i: (0, 0)),
            pl.BlockSpec((1, d), lambda i: (0, 0)),
        ],
        out_specs=pl.BlockSpec((row_block, d), lambda i: (i, 0)),
        out_shape=jax.ShapeDtypeStruct((m, d), jnp.float32),
    )(e, We, be.reshape(1, -1), gamma_e.reshape(1, -1), beta_e.reshape(1, -1))


def _sc_phase1(n, m, d, g16, src, dst, bh0, bh1, dh0, dh1, th0, th1,
               ce, e, zeros):
    f32 = jnp.float32
    hd = d // 2
    ept = m // _NSUB
    nch = ept // _CH
    assert nch % 2 == 0 and nch >= 4
    rows_per = (n // (8 * _NSUB)) * 8
    rows_tail = n - rows_per * _NSUB
    nq = hd // 16
    mesh = plsc.VectorSubcoreMesh(core_axis_name="c", subcore_axis_name="s")

    vbuf = pltpu.VMEM((_CH, hd), f32)
    ibuf = pltpu.VMEM((_CH,), jnp.int32)
    dsem = pltpu.SemaphoreType.DMA

    @functools.partial(
        pl.kernel,
        mesh=mesh,
        out_type=[jax.ShapeDtypeStruct((m, d), f32),
                  jax.ShapeDtypeStruct((m, hd), f32),
                  jax.ShapeDtypeStruct((m, hd), f32),
                  jax.ShapeDtypeStruct((n, hd), f32),
                  jax.ShapeDtypeStruct((n, hd), f32)],
        scratch_types=(
            [ibuf] * 6 +
            [vbuf] * 10 +
            [vbuf] * 6 +
            [pltpu.VMEM((16,), f32)] +
            [pltpu.VMEM_SHARED((n, hd), f32)] +
            [dsem] * 8
        ),
        compiler_params=_SC_PARAMS,
    )
    def sc_kernel(g_hbm, src_hbm, dst_hbm, bh0_h, bh1_h, dh0_h, dh1_h,
                  th0_h, th1_h, ce_h, e_h, zz_h,
                  eo_h, sg0_h, sg1_h, ph0_h, ph1_h,
                  is0, is1, id0, id1, ic0, ic1,
                  dh_0, dh_1, eh_0, eh_1, bh_0, bh_1, ce_0, ce_1, er_0, er_1,
                  sg_0, sg_1, ms_0, ms_1, eo_0, eo_1,
                  gv, acc,
                  si0, si1, sd0, sd1, sw0, sw1, ss0, ss1):
        c = lax.axis_index("c")
        s = lax.axis_index("s")
        r0 = s * rows_per
        pltpu.sync_copy(zz_h.at[pl.ds(r0, rows_per)],
                        acc.at[pl.ds(r0, rows_per)])
        if rows_tail:
            @pl.when(s == _NSUB - 1)
            def _():
                t0 = rows_per * _NSUB
                pltpu.sync_copy(zz_h.at[pl.ds(t0, rows_tail)],
                                acc.at[pl.ds(t0, rows_tail)])
        pltpu.sync_copy(g_hbm, gv)
        plsc.subcore_barrier()

        IS, ID, IC = [is0, is1], [id0, id1], [ic0, ic1]
        DH, EH, BH = [dh_0, dh_1], [eh_0, eh_1], [bh_0, bh_1]
        CE, ER = [ce_0, ce_1], [er_0, er_1]
        SG, MS, EO = [sg_0, sg_1], [ms_0, ms_1], [eo_0, eo_1]
        SI, SD = [si0, si1], [sd0, sd1]
        SW, SS = [sw0, sw1], [ss0, ss1]

        def run_half(bh_t, dh_t, th_t, sg_out, ph_t, col0):
            gval = gv[...]

            def base_of(gi):
                return s * ept + gi * _CH

            def issue_idx(gi, b):
                bb = base_of(gi)
                pltpu.async_copy(src_hbm.at[pl.ds(bb, _CH)], IS[b], SI[b])
                pltpu.async_copy(dst_hbm.at[pl.ds(bb, _CH)], ID[b], SI[b])

            def wait_idx(b):
                pltpu.make_async_copy(src_hbm.at[pl.ds(0, _CH)], IS[b],
                                      SI[b]).wait()
                pltpu.make_async_copy(dst_hbm.at[pl.ds(0, _CH)], ID[b],
                                      SI[b]).wait()

            def issue_data(gi, b):
                bb = base_of(gi)
                pltpu.async_copy(dh_t.at[IS[b]], DH[b], SD[b])
                pltpu.async_copy(th_t.at[ID[b]], EH[b], SD[b])
                pltpu.async_copy(bh_t.at[IS[b]], BH[b], SD[b])
                pltpu.async_copy(ce_h.at[pl.ds(bb, _CH), pl.ds(col0, hd)],
                                 CE[b], SD[b])
                pltpu.async_copy(e_h.at[pl.ds(bb, _CH), pl.ds(col0, hd)],
                                 ER[b], SD[b])

            def wait_data(b):
                pltpu.make_async_copy(dh_t.at[IS[b]], DH[b], SD[b]).wait()
                pltpu.make_async_copy(th_t.at[ID[b]], EH[b], SD[b]).wait()
                pltpu.make_async_copy(bh_t.at[IS[b]], BH[b], SD[b]).wait()
                pltpu.make_async_copy(
                    ce_h.at[pl.ds(0, _CH), pl.ds(col0, hd)], CE[b],
                    SD[b]).wait()
                pltpu.make_async_copy(
                    e_h.at[pl.ds(0, _CH), pl.ds(col0, hd)], ER[b],
                    SD[b]).wait()

            def copy_idx_sc(b):
                for i in range(_CH // 16):
                    sl = pl.ds(i * 16, 16)
                    IC[b][sl] = ID[b][sl]

            def compute(b):
                @pl.loop(0, _CH)
                def _(j):
                    for q in range(nq):
                        sl = pl.ds(q * 16, 16)
                        x = DH[b][j, sl] + EH[b][j, sl] + CE[b][j, sl]
                        sgv = 1.0 / (1.0 + jnp.exp(-x))
                        SG[b][j, sl] = sgv
                        MS[b][j, sl] = BH[b][j, sl] * sgv
                        EO[b][j, sl] = ER[b][j, sl] + gval * jnp.maximum(
                            x, 0.0)

            def issue_writes(gi, b):
                bb = base_of(gi)
                pltpu.async_copy(EO[b],
                                 eo_h.at[pl.ds(bb, _CH), pl.ds(col0, hd)],
                                 SW[b])
                pltpu.async_copy(SG[b], sg_out.at[pl.ds(bb, _CH)], SW[b])
                pltpu.async_copy(MS[b], acc.at[IC[b]], SS[b], add=True)

            def wait_writes(b):
                pltpu.make_async_copy(
                    EO[b], eo_h.at[pl.ds(0, _CH), pl.ds(col0, hd)],
                    SW[b]).wait()
                pltpu.make_async_copy(SG[b], sg_out.at[pl.ds(0, _CH)],
                                      SW[b]).wait()
                pltpu.make_async_copy(MS[b], acc.at[IC[b]], SS[b]).wait()

            issue_idx(0, 0)
            issue_idx(1, 1)
            wait_idx(0)
            issue_data(0, 0)

            @pl.loop(0, nch // 2)
            def _(t):
                for b in (0, 1):
                    @pl.when(t >= 1)
                    def _():
                        wait_writes(b)
                    wait_data(b)
                    copy_idx_sc(b)

                    @pl.when(t < nch // 2 - 1)
                    def _():
                        issue_idx(2 * t + b + 2, b)

                    if b == 0:
                        wait_idx(1)
                        issue_data(2 * t + 1, 1)
                    else:
                        @pl.when(t < nch // 2 - 1)
                        def _():
                            wait_idx(0)
                            issue_data(2 * t + 2, 0)
                    compute(b)
                    issue_writes(2 * t + b, b)

            wait_writes(0)
            wait_writes(1)

            plsc.subcore_barrier()
            pltpu.sync_copy(acc.at[pl.ds(r0, rows_per)],
                            ph_t.at[pl.ds(r0, rows_per)])
            if rows_tail:
                @pl.when(s == _NSUB - 1)
                def _():
                    t0 = rows_per * _NSUB
                    pltpu.sync_copy(acc.at[pl.ds(t0, rows_tail)],
                                    ph_t.at[pl.ds(t0, rows_tail)])

        @pl.when(c == 0)
        def _():
            run_half(bh0_h, dh0_h, th0_h, sg0_h, ph0_h, 0)

        @pl.when(c == 1)
        def _():
            run_half(bh1_h, dh1_h, th1_h, sg1_h, ph1_h, hd)

    return sc_kernel(g16, src, dst, bh0, bh1, dh0, dh1, th0, th1,
                     ce, e, zeros)


def _sc_phase2(n, m, d, dst, sg0, sg1, zeros):
    f32 = jnp.float32
    hd = d // 2
    ept = m // _NSUB
    nch = ept // _CH
    rows_per = (n // (8 * _NSUB)) * 8
    rows_tail = n - rows_per * _NSUB
    mesh = plsc.VectorSubcoreMesh(core_axis_name="c", subcore_axis_name="s")

    ch2 = _CH // 2
    nch2 = ept // ch2
    assert nch2 % 4 == 0 and nch2 >= 8
    ibuf = pltpu.VMEM((ch2,), jnp.int32)
    vbuf = pltpu.VMEM((ch2, hd), f32)
    dsem = pltpu.SemaphoreType.DMA

    @functools.partial(
        pl.kernel,
        mesh=mesh,
        out_type=[jax.ShapeDtypeStruct((n, hd), f32),
                  jax.ShapeDtypeStruct((n, hd), f32)],
        scratch_types=(
            [ibuf] * 4 +
            [vbuf] * 4 +
            [pltpu.VMEM_SHARED((n, hd), f32)] +
            [dsem] * 8
        ),
        compiler_params=_SC_PARAMS,
    )
    def sc_kernel(dst_hbm, sg0_h, sg1_h, zz_h, ps0_h, ps1_h,
                  id0, id1, id2, id3, sg_0, sg_1, sg_2, sg_3, acc,
                  sl0, sl1, sl2, sl3, ss0, ss1, ss2, ss3):
        c = lax.axis_index("c")
        s = lax.axis_index("s")
        r0 = s * rows_per
        pltpu.sync_copy(zz_h.at[pl.ds(r0, rows_per)],
                        acc.at[pl.ds(r0, rows_per)])
        if rows_tail:
            @pl.when(s == _NSUB - 1)
            def _():
                t0 = rows_per * _NSUB
                pltpu.sync_copy(zz_h.at[pl.ds(t0, rows_tail)],
                                acc.at[pl.ds(t0, rows_tail)])
        plsc.subcore_barrier()

        ID = [id0, id1, id2, id3]
        SGR = [sg_0, sg_1, sg_2, sg_3]
        SL = [sl0, sl1, sl2, sl3]
        SS = [ss0, ss1, ss2, ss3]

        def run_half(sg_t, ps_t):
            def issue_loads(gi, b):
                base = s * ept + gi * ch2
                pltpu.async_copy(dst_hbm.at[pl.ds(base, ch2)], ID[b], SL[b])
                pltpu.async_copy(sg_t.at[pl.ds(base, ch2)], SGR[b], SL[b])

            def wait_loads(b):
                pltpu.make_async_copy(dst_hbm.at[pl.ds(0, ch2)], ID[b],
                                      SL[b]).wait()
                pltpu.make_async_copy(sg_t.at[pl.ds(0, ch2)], SGR[b],
                                      SL[b]).wait()

            def wait_scatter(b):
                pltpu.make_async_copy(SGR[b], acc.at[ID[b]], SS[b]).wait()

            issue_loads(0, 0)
            issue_loads(1, 1)

            @pl.loop(0, nch2 // 4)
            def _(t):
                for b in range(4):
                    wait_loads(b)
                    pltpu.async_copy(SGR[b], acc.at[ID[b]], SS[b],
                                     add=True)
                    nb2 = (b + 2) % 4
                    if b < 2:
                        @pl.when(t >= 1)
                        def _():
                            wait_scatter(nb2)
                        issue_loads(4 * t + b + 2, nb2)
                    else:
                        wait_scatter(nb2)

                        @pl.when(t < nch2 // 4 - 1)
                        def _():
                            issue_loads(4 * t + b + 2, nb2)

            wait_scatter((nch2 - 2) % 4)
            wait_scatter((nch2 - 1) % 4)

            plsc.subcore_barrier()
            pltpu.sync_copy(acc.at[pl.ds(r0, rows_per)],
                            ps_t.at[pl.ds(r0, rows_per)])
            if rows_tail:
                @pl.when(s == _NSUB - 1)
                def _():
                    t0 = rows_per * _NSUB
                    pltpu.sync_copy(acc.at[pl.ds(t0, rows_tail)],
                                    ps_t.at[pl.ds(t0, rows_tail)])

        @pl.when(c == 0)
        def _():
            run_half(sg0_h, ps0_h)

        @pl.when(c == 1)
        def _():
            run_half(sg1_h, ps1_h)

    return sc_kernel(dst, sg0, sg1, zeros)


def _node_out_body(h_ref, ah_ref, ph0_ref, ph1_ref, ps0_ref, ps1_ref, g_ref,
                   out_ref):
    d = h_ref.shape[1]
    hd = d // 2
    g = g_ref[0, 0]
    rl = ph0_ref[...] / (ps0_ref[...] + 1e-6)
    rh = ph1_ref[...] / (ps1_ref[...] + 1e-6)
    out_ref[:, :hd] = h_ref[:, :hd] + g * jnp.maximum(ah_ref[:, :hd] + rl,
                                                      0.0)
    out_ref[:, hd:] = h_ref[:, hd:] + g * jnp.maximum(ah_ref[:, hd:] + rh,
                                                      0.0)


def _node_out(h, ah, ph0, ph1, ps0, ps1, g, row_block=2000):
    n, d = h.shape
    hd = d // 2
    grid = (n // row_block,)
    spec = pl.BlockSpec((row_block, d), lambda i: (i, 0))
    hspec = pl.BlockSpec((row_block, hd), lambda i: (i, 0))
    return pl.pallas_call(
        _node_out_body,
        grid=grid,
        in_specs=[spec, spec, hspec, hspec, hspec, hspec,
                  pl.BlockSpec((1, 1), lambda i: (0, 0))],
        out_specs=spec,
        out_shape=jax.ShapeDtypeStruct((n, d), jnp.float32),
    )(h, ah, ph0, ph1, ps0, ps1, g.reshape(1, 1))


def kernel(h, e, edge_index, Wn, bn, We, be, gamma_n, beta_n,
           gamma_e, beta_e, g):
    n, d = h.shape
    m = e.shape[0]
    hd = d // 2

    ah, bh0, bh1, dh0, dh1, th0, th1 = _node_proj(h, Wn, bn, gamma_n, beta_n)
    ce = _edge_proj(e, We, be, gamma_e, beta_e)

    src = edge_index[0]
    dst = edge_index[1]

    g16 = jnp.broadcast_to(g.astype(jnp.float32), (16,))
    zeros = jnp.zeros((n, hd), jnp.float32)
    e_out, sg0, sg1, ph0, ph1 = _sc_phase1(
        n, m, d, g16, src, dst, bh0, bh1, dh0, dh1, th0, th1, ce, e, zeros)
    ps0, ps1 = _sc_phase2(n, m, d, dst, sg0, sg1, zeros)

    h_out = _node_out(h, ah, ph0, ph1, ps0, ps1, g)
    return (h_out, e_out)


# speedup vs baseline: 1.5203x; 1.0011x over previous
import functools
import math

import jax
import jax.numpy as jnp
from jax import lax
from jax.experimental import pallas as pl
from jax.experimental.pallas import tpu as pltpu
from jax.experimental.pallas import tpu_sc as plsc

_EPS_BN = 1e-5
_INV_SQRT = 1.0 / math.sqrt(1.0 + _EPS_BN)

_CH = 80
_NSUB = 16

_SC_PARAMS = pltpu.CompilerParams(use_tc_tiling_on_sc=False)


def _node_proj_body(h_ref, wn_ref, bn_ref, gn_ref, btn_ref,
                    ah_ref, bh0_ref, bh1_ref, dh0_ref, dh1_ref,
                    th0_ref, th1_ref):
    d = h_ref.shape[1]
    hd = d // 2
    scale = gn_ref[0, :] * _INV_SQRT
    hb = h_ref[...] * scale[None, :] + btn_ref[0, :][None, :]
    H = lax.dot_general(hb, wn_ref[...], (((1,), (1,)), ((), ())),
                        preferred_element_type=jnp.float32)
    H = H + bn_ref[0, :][None, :]
    ah_ref[...] = H[:, :d]
    bh0_ref[...] = H[:, d:d + hd]
    bh1_ref[...] = H[:, d + hd:2 * d]
    dh0_ref[...] = H[:, 2 * d:2 * d + hd]
    dh1_ref[...] = H[:, 2 * d + hd:3 * d]
    th0_ref[...] = H[:, 3 * d:3 * d + hd]
    th1_ref[...] = H[:, 3 * d + hd:4 * d]


def _node_proj(h, Wn, bn, gamma_n, beta_n, row_block=2000):
    n, d = h.shape
    hd = d // 2
    grid = (n // row_block,)
    f32 = jnp.float32
    outs = [jax.ShapeDtypeStruct((n, d), f32)] + \
           [jax.ShapeDtypeStruct((n, hd), f32)] * 6
    return pl.pallas_call(
        _node_proj_body,
        grid=grid,
        in_specs=[
            pl.BlockSpec((row_block, d), lambda i: (i, 0)),
            pl.BlockSpec((4 * d, d), lambda i: (0, 0)),
            pl.BlockSpec((1, 4 * d), lambda i: (0, 0)),
            pl.BlockSpec((1, d), lambda i: (0, 0)),
            pl.BlockSpec((1, d), lambda i: (0, 0)),
        ],
        out_specs=[pl.BlockSpec((row_block, d), lambda i: (i, 0))] +
                  [pl.BlockSpec((row_block, hd), lambda i: (i, 0))] * 6,
        out_shape=outs,
    )(h, Wn, bn.reshape(1, -1), gamma_n.reshape(1, -1), beta_n.reshape(1, -1))


def _edge_proj_body(e_ref, we_ref, be_ref, ge_ref, bte_ref, ce_ref):
    scale = ge_ref[0, :] * _INV_SQRT
    eb = e_ref[...] * scale[None, :] + bte_ref[0, :][None, :]
    C = lax.dot_general(eb, we_ref[...], (((1,), (1,)), ((), ())),
                        preferred_element_type=jnp.float32)
    ce_ref[...] = C + be_ref[0, :][None, :]


def _edge_proj(e, We, be, gamma_e, beta_e, row_block=16000):
    m, d = e.shape
    grid = (m // row_block,)
    return pl.pallas_call(
        _edge_proj_body,
        grid=grid,
        in_specs=[
            pl.BlockSpec((row_block, d), lambda i: (i, 0)),
            pl.BlockSpec((d, d), lambda i: (0, 0)),
            pl.BlockSpec((1, d), lambda i: (0, 0)),
            pl.BlockSpec((1, d), lambda i: (0, 0)),
            pl.BlockSpec((1, d), lambda i: (0, 0)),
        ],
        out_specs=pl.BlockSpec((row_block, d), lambda i: (i, 0)),
        out_shape=jax.ShapeDtypeStruct((m, d), jnp.float32),
    )(e, We, be.reshape(1, -1), gamma_e.reshape(1, -1), beta_e.reshape(1, -1))


def _sc_phase1(n, m, d, g16, src, dst, bh0, bh1, dh0, dh1, th0, th1,
               ce, e, zeros):
    f32 = jnp.float32
    hd = d // 2
    ept = m // _NSUB
    nch = ept // _CH
    assert nch % 2 == 0 and nch >= 4
    rows_per = (n // (8 * _NSUB)) * 8
    rows_tail = n - rows_per * _NSUB
    nq = hd // 16
    mesh = plsc.VectorSubcoreMesh(core_axis_name="c", subcore_axis_name="s")

    vbuf = pltpu.VMEM((_CH, hd), f32)
    ibuf = pltpu.VMEM((_CH,), jnp.int32)
    dsem = pltpu.SemaphoreType.DMA

    @functools.partial(
        pl.kernel,
        mesh=mesh,
        out_type=[jax.ShapeDtypeStruct((m, d), f32),
                  jax.ShapeDtypeStruct((m, hd), f32),
                  jax.ShapeDtypeStruct((m, hd), f32),
                  jax.ShapeDtypeStruct((n, hd), f32),
                  jax.ShapeDtypeStruct((n, hd), f32)],
        scratch_types=(
            [ibuf] * 6 +
            [vbuf] * 10 +
            [vbuf] * 6 +
            [pltpu.VMEM((16,), f32)] +
            [pltpu.VMEM_SHARED((n, hd), f32)] +
            [dsem] * 8
        ),
        compiler_params=_SC_PARAMS,
    )
    def sc_kernel(g_hbm, src_hbm, dst_hbm, bh0_h, bh1_h, dh0_h, dh1_h,
                  th0_h, th1_h, ce_h, e_h, zz_h,
                  eo_h, sg0_h, sg1_h, ph0_h, ph1_h,
                  is0, is1, id0, id1, ic0, ic1,
                  dh_0, dh_1, eh_0, eh_1, bh_0, bh_1, ce_0, ce_1, er_0, er_1,
                  sg_0, sg_1, ms_0, ms_1, eo_0, eo_1,
                  gv, acc,
                  si0, si1, sd0, sd1, sw0, sw1, ss0, ss1):
        c = lax.axis_index("c")
        s = lax.axis_index("s")
        r0 = s * rows_per
        pltpu.sync_copy(zz_h.at[pl.ds(r0, rows_per)],
                        acc.at[pl.ds(r0, rows_per)])
        if rows_tail:
            @pl.when(s == _NSUB - 1)
            def _():
                t0 = rows_per * _NSUB
                pltpu.sync_copy(zz_h.at[pl.ds(t0, rows_tail)],
                                acc.at[pl.ds(t0, rows_tail)])
        pltpu.sync_copy(g_hbm, gv)
        plsc.subcore_barrier()

        IS, ID, IC = [is0, is1], [id0, id1], [ic0, ic1]
        DH, EH, BH = [dh_0, dh_1], [eh_0, eh_1], [bh_0, bh_1]
        CE, ER = [ce_0, ce_1], [er_0, er_1]
        SG, MS, EO = [sg_0, sg_1], [ms_0, ms_1], [eo_0, eo_1]
        SI, SD = [si0, si1], [sd0, sd1]
        SW, SS = [sw0, sw1], [ss0, ss1]

        def run_half(bh_t, dh_t, th_t, sg_out, ph_t, col0):
            gval = gv[...]

            def base_of(gi):
                return s * ept + gi * _CH

            def issue_idx(gi, b):
                bb = base_of(gi)
                pltpu.async_copy(src_hbm.at[pl.ds(bb, _CH)], IS[b], SI[b])
                pltpu.async_copy(dst_hbm.at[pl.ds(bb, _CH)], ID[b], SI[b])

            def wait_idx(b):
                pltpu.make_async_copy(src_hbm.at[pl.ds(0, _CH)], IS[b],
                                      SI[b]).wait()
                pltpu.make_async_copy(dst_hbm.at[pl.ds(0, _CH)], ID[b],
                                      SI[b]).wait()

            def issue_data(gi, b):
                bb = base_of(gi)
                pltpu.async_copy(dh_t.at[IS[b]], DH[b], SD[b])
                pltpu.async_copy(th_t.at[ID[b]], EH[b], SD[b])
                pltpu.async_copy(bh_t.at[IS[b]], BH[b], SD[b])
                pltpu.async_copy(ce_h.at[pl.ds(bb, _CH), pl.ds(col0, hd)],
                                 CE[b], SD[b])
                pltpu.async_copy(e_h.at[pl.ds(bb, _CH), pl.ds(col0, hd)],
                                 ER[b], SD[b])

            def wait_data(b):
                pltpu.make_async_copy(dh_t.at[IS[b]], DH[b], SD[b]).wait()
                pltpu.make_async_copy(th_t.at[ID[b]], EH[b], SD[b]).wait()
                pltpu.make_async_copy(bh_t.at[IS[b]], BH[b], SD[b]).wait()
                pltpu.make_async_copy(
                    ce_h.at[pl.ds(0, _CH), pl.ds(col0, hd)], CE[b],
                    SD[b]).wait()
                pltpu.make_async_copy(
                    e_h.at[pl.ds(0, _CH), pl.ds(col0, hd)], ER[b],
                    SD[b]).wait()

            def copy_idx_sc(b):
                for i in range(_CH // 16):
                    sl = pl.ds(i * 16, 16)
                    IC[b][sl] = ID[b][sl]

            def compute(b):
                @pl.loop(0, _CH)
                def _(j):
                    for q in range(nq):
                        sl = pl.ds(q * 16, 16)
                        x = DH[b][j, sl] + EH[b][j, sl] + CE[b][j, sl]
                        sgv = 1.0 / (1.0 + jnp.exp(-x))
                        SG[b][j, sl] = sgv
                        MS[b][j, sl] = BH[b][j, sl] * sgv
                        EO[b][j, sl] = ER[b][j, sl] + gval * jnp.maximum(
                            x, 0.0)

            def issue_writes(gi, b):
                bb = base_of(gi)
                pltpu.async_copy(EO[b],
                                 eo_h.at[pl.ds(bb, _CH), pl.ds(col0, hd)],
                                 SW[b])
                pltpu.async_copy(SG[b], sg_out.at[pl.ds(bb, _CH)], SW[b])
                pltpu.async_copy(MS[b], acc.at[IC[b]], SS[b], add=True)

            def wait_writes(b):
                pltpu.make_async_copy(
                    EO[b], eo_h.at[pl.ds(0, _CH), pl.ds(col0, hd)],
                    SW[b]).wait()
                pltpu.make_async_copy(SG[b], sg_out.at[pl.ds(0, _CH)],
                                      SW[b]).wait()
                pltpu.make_async_copy(MS[b], acc.at[IC[b]], SS[b]).wait()

            issue_idx(0, 0)
            issue_idx(1, 1)
            wait_idx(0)
            issue_data(0, 0)

            @pl.loop(0, nch // 2)
            def _(t):
                for b in (0, 1):
                    @pl.when(t >= 1)
                    def _():
                        wait_writes(b)
                    wait_data(b)
                    copy_idx_sc(b)

                    @pl.when(t < nch // 2 - 1)
                    def _():
                        issue_idx(2 * t + b + 2, b)

                    if b == 0:
                        wait_idx(1)
                        issue_data(2 * t + 1, 1)
                    else:
                        @pl.when(t < nch // 2 - 1)
                        def _():
                            wait_idx(0)
                            issue_data(2 * t + 2, 0)
                    compute(b)
                    issue_writes(2 * t + b, b)

            wait_writes(0)
            wait_writes(1)

            plsc.subcore_barrier()
            pltpu.sync_copy(acc.at[pl.ds(r0, rows_per)],
                            ph_t.at[pl.ds(r0, rows_per)])
            if rows_tail:
                @pl.when(s == _NSUB - 1)
                def _():
                    t0 = rows_per * _NSUB
                    pltpu.sync_copy(acc.at[pl.ds(t0, rows_tail)],
                                    ph_t.at[pl.ds(t0, rows_tail)])

        @pl.when(c == 0)
        def _():
            run_half(bh0_h, dh0_h, th0_h, sg0_h, ph0_h, 0)

        @pl.when(c == 1)
        def _():
            run_half(bh1_h, dh1_h, th1_h, sg1_h, ph1_h, hd)

    return sc_kernel(g16, src, dst, bh0, bh1, dh0, dh1, th0, th1,
                     ce, e, zeros)


def _sc_phase2(n, m, d, dst, sg0, sg1, zeros):
    f32 = jnp.float32
    hd = d // 2
    ept = m // _NSUB
    nch = ept // _CH
    rows_per = (n // (8 * _NSUB)) * 8
    rows_tail = n - rows_per * _NSUB
    mesh = plsc.VectorSubcoreMesh(core_axis_name="c", subcore_axis_name="s")

    ch2 = _CH // 2
    nch2 = ept // ch2
    assert nch2 % 4 == 0 and nch2 >= 8
    ibuf = pltpu.VMEM((ch2,), jnp.int32)
    vbuf = pltpu.VMEM((ch2, hd), f32)
    dsem = pltpu.SemaphoreType.DMA

    @functools.partial(
        pl.kernel,
        mesh=mesh,
        out_type=[jax.ShapeDtypeStruct((n, hd), f32),
                  jax.ShapeDtypeStruct((n, hd), f32)],
        scratch_types=(
            [ibuf] * 4 +
            [vbuf] * 4 +
            [pltpu.VMEM_SHARED((n, hd), f32)] +
            [dsem] * 8
        ),
        compiler_params=_SC_PARAMS,
    )
    def sc_kernel(dst_hbm, sg0_h, sg1_h, zz_h, ps0_h, ps1_h,
                  id0, id1, id2, id3, sg_0, sg_1, sg_2, sg_3, acc,
                  sl0, sl1, sl2, sl3, ss0, ss1, ss2, ss3):
        c = lax.axis_index("c")
        s = lax.axis_index("s")
        r0 = s * rows_per
        pltpu.sync_copy(zz_h.at[pl.ds(r0, rows_per)],
                        acc.at[pl.ds(r0, rows_per)])
        if rows_tail:
            @pl.when(s == _NSUB - 1)
            def _():
                t0 = rows_per * _NSUB
                pltpu.sync_copy(zz_h.at[pl.ds(t0, rows_tail)],
                                acc.at[pl.ds(t0, rows_tail)])
        plsc.subcore_barrier()

        ID = [id0, id1, id2, id3]
        SGR = [sg_0, sg_1, sg_2, sg_3]
        SL = [sl0, sl1, sl2, sl3]
        SS = [ss0, ss1, ss2, ss3]

        def run_half(sg_t, ps_t):
            def issue_loads(gi, b):
                base = s * ept + gi * ch2
                pltpu.async_copy(dst_hbm.at[pl.ds(base, ch2)], ID[b], SL[b])
                pltpu.async_copy(sg_t.at[pl.ds(base, ch2)], SGR[b], SL[b])

            def wait_loads(b):
                pltpu.make_async_copy(dst_hbm.at[pl.ds(0, ch2)], ID[b],
                                      SL[b]).wait()
                pltpu.make_async_copy(sg_t.at[pl.ds(0, ch2)], SGR[b],
                                      SL[b]).wait()

            def wait_scatter(b):
                pltpu.make_async_copy(SGR[b], acc.at[ID[b]], SS[b]).wait()

            issue_loads(0, 0)
            issue_loads(1, 1)

            @pl.loop(0, nch2 // 4)
            def _(t):
                for b in range(4):
                    wait_loads(b)
                    pltpu.async_copy(SGR[b], acc.at[ID[b]], SS[b],
                                     add=True)
                    nb2 = (b + 2) % 4
                    if b < 2:
                        @pl.when(t >= 1)
                        def _():
                            wait_scatter(nb2)
                        issue_loads(4 * t + b + 2, nb2)
                    else:
                        wait_scatter(nb2)

                        @pl.when(t < nch2 // 4 - 1)
                        def _():
                            issue_loads(4 * t + b + 2, nb2)

            wait_scatter((nch2 - 2) % 4)
            wait_scatter((nch2 - 1) % 4)

            plsc.subcore_barrier()
            pltpu.sync_copy(acc.at[pl.ds(r0, rows_per)],
                            ps_t.at[pl.ds(r0, rows_per)])
            if rows_tail:
                @pl.when(s == _NSUB - 1)
                def _():
                    t0 = rows_per * _NSUB
                    pltpu.sync_copy(acc.at[pl.ds(t0, rows_tail)],
                                    ps_t.at[pl.ds(t0, rows_tail)])

        @pl.when(c == 0)
        def _():
            run_half(sg0_h, ps0_h)

        @pl.when(c == 1)
        def _():
            run_half(sg1_h, ps1_h)

    return sc_kernel(dst, sg0, sg1, zeros)


def _node_out_body(h_ref, ah_ref, ph0_ref, ph1_ref, ps0_ref, ps1_ref, g_ref,
                   out_ref):
    d = h_ref.shape[1]
    hd = d // 2
    g = g_ref[0, 0]
    rl = ph0_ref[...] / (ps0_ref[...] + 1e-6)
    rh = ph1_ref[...] / (ps1_ref[...] + 1e-6)
    out_ref[:, :hd] = h_ref[:, :hd] + g * jnp.maximum(ah_ref[:, :hd] + rl,
                                                      0.0)
    out_ref[:, hd:] = h_ref[:, hd:] + g * jnp.maximum(ah_ref[:, hd:] + rh,
                                                      0.0)


def _node_out(h, ah, ph0, ph1, ps0, ps1, g, row_block=2000):
    n, d = h.shape
    hd = d // 2
    grid = (n // row_block,)
    spec = pl.BlockSpec((row_block, d), lambda i: (i, 0))
    hspec = pl.BlockSpec((row_block, hd), lambda i: (i, 0))
    return pl.pallas_call(
        _node_out_body,
        grid=grid,
        in_specs=[spec, spec, hspec, hspec, hspec, hspec,
                  pl.BlockSpec((1, 1), lambda i: (0, 0))],
        out_specs=spec,
        out_shape=jax.ShapeDtypeStruct((n, d), jnp.float32),
    )(h, ah, ph0, ph1, ps0, ps1, g.reshape(1, 1))


def kernel(h, e, edge_index, Wn, bn, We, be, gamma_n, beta_n,
           gamma_e, beta_e, g):
    n, d = h.shape
    m = e.shape[0]
    hd = d // 2

    ah, bh0, bh1, dh0, dh1, th0, th1 = _node_proj(h, Wn, bn, gamma_n, beta_n)
    ce = _edge_proj(e, We, be, gamma_e, beta_e)

    src = edge_index[0]
    dst = edge_index[1]

    g16 = jnp.broadcast_to(g.astype(jnp.float32), (16,))
    zeros = jnp.zeros((n, hd), jnp.float32)
    e_out, sg0, sg1, ph0, ph1 = _sc_phase1(
        n, m, d, g16, src, dst, bh0, bh1, dh0, dh1, th0, th1, ce, e, zeros)
    ps0, ps1 = _sc_phase2(n, m, d, dst, sg0, sg1, zeros)

    h_out = _node_out(h, ah, ph0, ph1, ps0, ps1, g)
    return (h_out, e_out)
